# baseline (speedup 1.0000x reference)
import jax
import jax.numpy as jnp
from jax.experimental import pallas as pl
from jax.experimental.pallas import tpu as pltpu

N = 50000
F = 128
T = 32
S = 32
B = 128

BLOCK_N = 1000
NB = N // BLOCK_N


def _ect_kernel(x_ref, v_ref, batch_ref, lin_ref, out_ref, acc_ref, cnt_ref):
    i = pl.program_id(0)

    @pl.when(i == 0)
    def _():
        acc_ref[...] = jnp.zeros_like(acc_ref)
        cnt_ref[...] = jnp.zeros_like(cnt_ref)

    xb = x_ref[...]
    vv = v_ref[...]
    nht = jax.lax.dot_general(
        vv, xb, (((0,), (1,)), ((), ())), preferred_element_type=jnp.float32
    )
    b2 = 100.0 * nht
    bflat = jnp.broadcast_to(b2[None, :, :], (S, T, BLOCK_N)).reshape(S * T, BLOCK_N)
    z = lin_ref[...] - bflat
    th = jnp.tanh(z)

    bcol = batch_ref[0].astype(jnp.float32)
    iota = jax.lax.broadcasted_iota(jnp.int32, (BLOCK_N, B), 1).astype(jnp.float32)
    onehot = (iota == bcol).astype(jnp.float32)

    acc_ref[...] += jnp.dot(th, onehot, preferred_element_type=jnp.float32)
    cnt_ref[0:1, :] += jnp.sum(onehot, axis=0, keepdims=True)

    @pl.when(i == NB - 1)
    def _():
        acc_t = acc_ref[...].T
        cnt_t = cnt_ref[0:1, :].T
        out_ref[...] = 0.5 * (acc_t + cnt_t)


@jax.jit
def kernel(x, batch, v, lin):
    lin_col = 100.0 * jnp.broadcast_to(lin.reshape(S, 1, 1), (S, T, 1)).reshape(S * T, 1)
    batch_col = batch.reshape(NB, BLOCK_N, 1)

    out = pl.pallas_call(
        _ect_kernel,
        grid=(NB,),
        in_specs=[
            pl.BlockSpec((BLOCK_N, F), lambda i: (i, 0)),
            pl.BlockSpec((F, T), lambda i: (0, 0)),
            pl.BlockSpec((1, BLOCK_N, 1), lambda i: (i, 0, 0)),
            pl.BlockSpec((S * T, 1), lambda i: (0, 0)),
        ],
        out_specs=pl.BlockSpec((B, S * T), lambda i: (0, 0)),
        out_shape=jax.ShapeDtypeStruct((B, S * T), jnp.float32),
        scratch_shapes=[
            pltpu.VMEM((S * T, B), jnp.float32),
            pltpu.VMEM((8, B), jnp.float32),
        ],
    )(x, v, batch_col, lin_col)

    return out.reshape(B, S, T)


# speedup vs baseline: 1.3039x; 1.3039x over previous
import jax
import jax.numpy as jnp
from jax.experimental import pallas as pl
from jax.experimental.pallas import tpu as pltpu

N = 50000
F = 128
T = 32
S = 32
B = 128

BLOCK_N = 5000
NB = N // BLOCK_N


def _ect_kernel(x_ref, v_ref, batch_ref, lin_ref, out_ref, acc_ref, cnt_ref,
                th_ref):
    i = pl.program_id(0)

    @pl.when(i == 0)
    def _():
        acc_ref[...] = jnp.zeros_like(acc_ref)
        cnt_ref[...] = jnp.zeros_like(cnt_ref)

    xb = x_ref[...]
    vv = v_ref[...]
    nht = jax.lax.dot_general(
        vv, xb, (((0,), (1,)), ((), ())), preferred_element_type=jnp.float32
    )
    b2 = 100.0 * nht

    for s in range(S):
        c_s = lin_ref[s, 0]
        th_ref[s * T:(s + 1) * T, :] = jnp.tanh(c_s - b2)

    bcol = batch_ref[0].astype(jnp.float32)
    iota = jax.lax.broadcasted_iota(jnp.int32, (BLOCK_N, B), 1).astype(jnp.float32)
    onehot = (iota == bcol).astype(jnp.float32)

    acc_ref[...] += jnp.dot(th_ref[...], onehot, preferred_element_type=jnp.float32)
    cnt_ref[0:1, :] += jnp.sum(onehot, axis=0, keepdims=True)

    @pl.when(i == NB - 1)
    def _():
        acc_t = acc_ref[...].T
        cnt_t = cnt_ref[0:1, :].T
        out_ref[...] = 0.5 * (acc_t + cnt_t)


@jax.jit
def kernel(x, batch, v, lin):
    lin_col = 100.0 * lin.reshape(S, 1)
    batch_col = batch.reshape(NB, BLOCK_N, 1)

    out = pl.pallas_call(
        _ect_kernel,
        grid=(NB,),
        in_specs=[
            pl.BlockSpec((BLOCK_N, F), lambda i: (i, 0)),
            pl.BlockSpec((F, T), lambda i: (0, 0)),
            pl.BlockSpec((1, BLOCK_N, 1), lambda i: (i, 0, 0)),
            pl.BlockSpec((S, 1), lambda i: (0, 0)),
        ],
        out_specs=pl.BlockSpec((B, S * T), lambda i: (0, 0)),
        out_shape=jax.ShapeDtypeStruct((B, S * T), jnp.float32),
        scratch_shapes=[
            pltpu.VMEM((S * T, B), jnp.float32),
            pltpu.VMEM((8, B), jnp.float32),
            pltpu.VMEM((S * T, BLOCK_N), jnp.float32),
        ],
    )(x, v, batch_col, lin_col)

    return out.reshape(B, S, T)


# speedup vs baseline: 2.0750x; 1.5913x over previous
import jax
import jax.numpy as jnp
from jax.experimental import pallas as pl
from jax.experimental.pallas import tpu as pltpu

N = 50000
F = 128
T = 32
S = 32
B = 128

BLOCK_N = 5000
NB = N // BLOCK_N


def _ect_kernel(x_ref, v_ref, batch_ref, lin_ref, out_ref, acc_ref, cnt_ref,
                th_ref):
    i = pl.program_id(0)

    @pl.when(i == 0)
    def _():
        acc_ref[...] = jnp.zeros_like(acc_ref)
        cnt_ref[...] = jnp.zeros_like(cnt_ref)

    xb = x_ref[...]
    vv = v_ref[...]
    nht = jax.lax.dot_general(
        vv, xb, (((0,), (1,)), ((), ())), preferred_element_type=jnp.float32
    )
    b2 = 100.0 * nht

    for s in range(S):
        c_s = lin_ref[0, s]
        th_ref[s * T:(s + 1) * T, :] = jnp.tanh(c_s - b2)

    brow = batch_ref[0].astype(jnp.float32)
    iota = jax.lax.broadcasted_iota(jnp.int32, (B, BLOCK_N), 0).astype(jnp.float32)
    onehot_t = (iota == brow).astype(jnp.float32)

    acc_ref[...] += jax.lax.dot_general(
        th_ref[...], onehot_t, (((1,), (1,)), ((), ())),
        preferred_element_type=jnp.float32,
    )
    cnt_ref[:, 0:1] += jnp.sum(onehot_t, axis=1, keepdims=True)

    @pl.when(i == NB - 1)
    def _():
        acc_t = acc_ref[...].T
        out_ref[...] = 0.5 * (acc_t + cnt_ref[:, 0:1])


@jax.jit
def kernel(x, batch, v, lin):
    lin_row = 100.0 * lin.reshape(1, S)
    batch_row = batch.reshape(NB, 1, BLOCK_N)

    out = pl.pallas_call(
        _ect_kernel,
        grid=(NB,),
        in_specs=[
            pl.BlockSpec((BLOCK_N, F), lambda i: (i, 0)),
            pl.BlockSpec((F, T), lambda i: (0, 0)),
            pl.BlockSpec((1, 1, BLOCK_N), lambda i: (i, 0, 0)),
            pl.BlockSpec((1, S), lambda i: (0, 0)),
        ],
        out_specs=pl.BlockSpec((B, S * T), lambda i: (0, 0)),
        out_shape=jax.ShapeDtypeStruct((B, S * T), jnp.float32),
        scratch_shapes=[
            pltpu.VMEM((S * T, B), jnp.float32),
            pltpu.VMEM((B, 8), jnp.float32),
            pltpu.VMEM((S * T, BLOCK_N), jnp.float32),
        ],
    )(x, v, batch_row, lin_row)

    return out.reshape(B, S, T)
